# baseline (speedup 1.0000x reference)
import functools

import jax
import jax.numpy as jnp
from jax import lax
from jax.experimental import pallas as pl
from jax.experimental.pallas import tpu as pltpu
from jax.experimental.pallas import tpu_sc as plsc

N_TOK = 8192
K = 8192
D = 256
TM = 256
N_TILES = N_TOK // TM
KC = 4096

SC_NC = 2
SC_NS = 16
SC_NW = SC_NC * SC_NS
CHUNK = 128
CHUNKS_TOTAL = N_TOK // CHUNK
CHUNKS_PER_W = CHUNKS_TOTAL // SC_NW


KSUB = 2048


def _dist_argmax_body(x_ref, emb_ref, idx_ref):
    xt = x_ref[...]
    ms, idxs = [], []
    for c in range(K // KSUB):
        dc = lax.dot_general(
            xt, emb_ref[c * KSUB:(c + 1) * KSUB, :],
            (((1,), (1,)), ((), ())),
            preferred_element_type=jnp.float32)
        m = jnp.max(dc, axis=1, keepdims=True)
        iota = lax.broadcasted_iota(jnp.int32, dc.shape, 1)
        i = jnp.min(jnp.where(dc == m, iota, jnp.int32(2**30)),
                    axis=1, keepdims=True) + c * KSUB
        ms.append(m)
        idxs.append(i)
    per = KC // KSUB
    cms, cidxs = [], []
    for b in range(K // KC):
        mv, iv = ms[b * per], idxs[b * per]
        for c in range(b * per + 1, (b + 1) * per):
            hit = ms[c] > mv
            iv = jnp.where(hit, idxs[c], iv)
            mv = jnp.maximum(mv, ms[c])
        cms.append(mv)
        cidxs.append(iv)
    acc_v = cms[0].astype(jnp.bfloat16).astype(jnp.float32)
    acc_i = cidxs[0]
    for b in range(1, K // KC):
        take = cms[b] > acc_v
        acc_v = jnp.where(take, cms[b].astype(jnp.bfloat16).astype(jnp.float32),
                          acc_v)
        acc_i = jnp.where(take, cidxs[b], acc_i)
    idx_ref[0, 0, :] = acc_i[:, 0]


def _compute_indices(fl, en):
    out = pl.pallas_call(
        _dist_argmax_body,
        grid=(N_TILES,),
        in_specs=[
            pl.BlockSpec((TM, D), lambda i: (i, 0)),
            pl.BlockSpec((K, D), lambda i: (0, 0)),
        ],
        out_specs=pl.BlockSpec((1, 1, TM), lambda i: (i, 0, 0)),
        out_shape=jax.ShapeDtypeStruct((N_TILES, 1, TM), jnp.int32),
    )(fl, en)
    return out.reshape(N_TOK)


def _gather_body(table_hbm, idx_hbm, out_hbm, idx_v, rows_v, sem):
    wid = lax.axis_index("s") * SC_NC + lax.axis_index("c")
    for b in range(CHUNKS_PER_W):
        chunk = wid * CHUNKS_PER_W + b
        pltpu.sync_copy(idx_hbm.at[chunk], idx_v)
        pltpu.async_copy(table_hbm.at[idx_v], rows_v, sem).wait()
        pltpu.sync_copy(rows_v, out_hbm.at[pl.ds(chunk * CHUNK, CHUNK)])


def _gather_rows(embeddings, idx_flat):
    idx2d = idx_flat.reshape(CHUNKS_TOTAL, CHUNK)
    mesh = plsc.VectorSubcoreMesh(core_axis_name="c", subcore_axis_name="s")
    k = functools.partial(
        pl.kernel,
        out_type=jax.ShapeDtypeStruct((N_TOK, D), jnp.float32),
        mesh=mesh,
        scratch_types=[
            pltpu.VMEM((CHUNK,), jnp.int32),
            pltpu.VMEM((CHUNK, D), jnp.float32),
            pltpu.SemaphoreType.DMA,
        ],
    )(_gather_body)
    return k(embeddings, idx2d)


def kernel(x, embeddings):
    shape = x.shape
    flatten = x.reshape(-1, shape[-1])
    fn = jnp.linalg.norm(flatten, axis=-1, keepdims=True)
    fl = flatten / jnp.maximum(fn, 1e-12)
    en_norm = jnp.linalg.norm(embeddings, axis=-1, keepdims=True)
    en = embeddings / jnp.maximum(en_norm, 1e-12)
    idx_flat = _compute_indices(fl, en)
    quantized = _gather_rows(embeddings, idx_flat)
    return quantized.reshape(shape), idx_flat.reshape(shape[:-1])


# speedup vs baseline: 1.0467x; 1.0467x over previous
import functools

import jax
import jax.numpy as jnp
from jax import lax
from jax.experimental import pallas as pl
from jax.experimental.pallas import tpu as pltpu
from jax.experimental.pallas import tpu_sc as plsc

N_TOK = 8192
K = 8192
D = 256
TM = 256
N_TILES = N_TOK // TM
KC = 4096

SC_NC = 2
SC_NS = 16
SC_NW = SC_NC * SC_NS
CHUNK = 128
CHUNKS_TOTAL = N_TOK // CHUNK
CHUNKS_PER_W = CHUNKS_TOTAL // SC_NW


KSUB = 4096


def _dist_argmax_body(x_ref, emb_ref, idx_ref):
    xt = x_ref[...]
    ms, idxs = [], []
    for c in range(K // KSUB):
        dc = lax.dot_general(
            xt, emb_ref[c * KSUB:(c + 1) * KSUB, :],
            (((1,), (1,)), ((), ())),
            preferred_element_type=jnp.float32)
        m = jnp.max(dc, axis=1, keepdims=True)
        iota = lax.broadcasted_iota(jnp.int32, dc.shape, 1)
        i = jnp.min(jnp.where(dc == m, iota, jnp.int32(2**30)),
                    axis=1, keepdims=True) + c * KSUB
        ms.append(m)
        idxs.append(i)
    per = KC // KSUB
    cms, cidxs = [], []
    for b in range(K // KC):
        mv, iv = ms[b * per], idxs[b * per]
        for c in range(b * per + 1, (b + 1) * per):
            hit = ms[c] > mv
            iv = jnp.where(hit, idxs[c], iv)
            mv = jnp.maximum(mv, ms[c])
        cms.append(mv)
        cidxs.append(iv)
    acc_v = cms[0].astype(jnp.bfloat16).astype(jnp.float32)
    acc_i = cidxs[0]
    for b in range(1, K // KC):
        take = cms[b] > acc_v
        acc_v = jnp.where(take, cms[b].astype(jnp.bfloat16).astype(jnp.float32),
                          acc_v)
        acc_i = jnp.where(take, cidxs[b], acc_i)
    idx_ref[0, 0, :] = acc_i[:, 0]


def _compute_indices(fl, en):
    out = pl.pallas_call(
        _dist_argmax_body,
        grid=(N_TILES,),
        in_specs=[
            pl.BlockSpec((TM, D), lambda i: (i, 0)),
            pl.BlockSpec((K, D), lambda i: (0, 0)),
        ],
        out_specs=pl.BlockSpec((1, 1, TM), lambda i: (i, 0, 0)),
        out_shape=jax.ShapeDtypeStruct((N_TILES, 1, TM), jnp.int32),
    )(fl, en)
    return out.reshape(N_TOK)


def _gather_body(table_hbm, idx_hbm, out_hbm, idx_v, rows_v, sem):
    wid = lax.axis_index("s") * SC_NC + lax.axis_index("c")
    for b in range(CHUNKS_PER_W):
        chunk = wid * CHUNKS_PER_W + b
        pltpu.sync_copy(idx_hbm.at[chunk], idx_v)
        pltpu.async_copy(table_hbm.at[idx_v], rows_v, sem).wait()
        pltpu.sync_copy(rows_v, out_hbm.at[pl.ds(chunk * CHUNK, CHUNK)])


def _gather_rows(embeddings, idx_flat):
    idx2d = idx_flat.reshape(CHUNKS_TOTAL, CHUNK)
    mesh = plsc.VectorSubcoreMesh(core_axis_name="c", subcore_axis_name="s")
    k = functools.partial(
        pl.kernel,
        out_type=jax.ShapeDtypeStruct((N_TOK, D), jnp.float32),
        mesh=mesh,
        scratch_types=[
            pltpu.VMEM((CHUNK,), jnp.int32),
            pltpu.VMEM((CHUNK, D), jnp.float32),
            pltpu.SemaphoreType.DMA,
        ],
    )(_gather_body)
    return k(embeddings, idx2d)


def kernel(x, embeddings):
    shape = x.shape
    flatten = x.reshape(-1, shape[-1])
    fn = jnp.linalg.norm(flatten, axis=-1, keepdims=True)
    fl = flatten / jnp.maximum(fn, 1e-12)
    en_norm = jnp.linalg.norm(embeddings, axis=-1, keepdims=True)
    en = embeddings / jnp.maximum(en_norm, 1e-12)
    idx_flat = _compute_indices(fl, en)
    quantized = _gather_rows(embeddings, idx_flat)
    return quantized.reshape(shape), idx_flat.reshape(shape[:-1])
